# baseline (speedup 1.0000x reference)
import functools

import jax
import jax.numpy as jnp
from jax import lax
from jax.experimental import pallas as pl
from jax.experimental.pallas import tpu as pltpu
from jax.experimental.pallas import tpu_sc as plsc

VOCAB = 1_000_000
EMBED_DIM = 32
NBUF = 2
LANE = 16


@functools.lru_cache(maxsize=None)
def _build_fmt(vocab_pad: int):
    info = plsc.get_sparse_core_info()
    nc, ns = info.num_cores, info.num_subcores
    nw = nc * ns
    nj = vocab_pad // 128
    ncomp8 = EMBED_DIM // 8
    jblk = 2
    nbuf = 6
    nfetch = -(-(-(-nj // jblk)) // nw)
    nfetch += (-nfetch) % nbuf
    n_groups = nfetch // nbuf
    toks = jblk * 128

    mesh = plsc.VectorSubcoreMesh(core_axis_name="c", subcore_axis_name="s")

    @functools.partial(
        pl.kernel,
        mesh=mesh,
        out_type=jax.ShapeDtypeStruct((vocab_pad, EMBED_DIM), jnp.float32),
        scratch_types=[
            pltpu.VMEM((nbuf, ncomp8, jblk, 8, 128), jnp.float32),
            pltpu.VMEM((nbuf, toks, EMBED_DIM + 1), jnp.float32),
            pltpu.SemaphoreType.DMA((nbuf,)),
            pltpu.SemaphoreType.DMA((nbuf,)),
        ],
        compiler_params=pltpu.CompilerParams(
            use_tc_tiling_on_sc=False, needs_layout_passes=False),
    )
    def fmt(w4_hbm, out_hbm, bin_v, obuf, isem, osem):
        wid = lax.axis_index("s") * nc + lax.axis_index("c")
        iota16 = jax.lax.iota(jnp.int32, LANE)

        def j_of(i):
            return jnp.minimum((wid * nfetch + i) * jblk, nj - jblk)

        def fetch(i, slot):
            return pltpu.make_async_copy(
                w4_hbm.at[:, pl.ds(j_of(i), jblk)], bin_v.at[slot],
                isem.at[slot])

        def writeback(i, slot):
            return pltpu.make_async_copy(
                obuf.at[slot, :, pl.ds(0, EMBED_DIM)],
                out_hbm.at[pl.ds(j_of(i) * 128, toks)],
                osem.at[slot])

        def transpose(slot):
            src = bin_v.at[slot]
            dst = obuf.at[slot]

            @plsc.parallel_loop(0, toks // LANE, 1, unroll=4)
            def _(q):
                jl = q // (128 // LANE)
                lb = q % (128 // LANE)
                ridx = q * LANE + iota16
                for c in range(EMBED_DIM):
                    cidx = jnp.full((LANE,), c, jnp.int32)
                    vals = src[c // 8, jl, c % 8, pl.ds(lb * LANE, LANE)]
                    plsc.store_scatter(dst, [ridx, cidx], vals)

        for slot in range(nbuf):
            fetch(slot, slot).start()

        def group(g, _):
            for slot in range(nbuf):
                i = g * nbuf + slot
                fetch(i, slot).wait()

                @pl.when(i >= nbuf)
                def _():
                    writeback(i, slot).wait()

                transpose(slot)
                fetch(i + nbuf, slot).start()
                writeback(i, slot).start()
            return 0

        lax.fori_loop(0, n_groups - 1, group, 0)

        for slot in range(nbuf):
            i = (n_groups - 1) * nbuf + slot
            fetch(i, slot).wait()
            writeback(i, slot).wait()
            transpose(slot)
            writeback(i, slot).start()
        for slot in range(nbuf):
            writeback(0, slot).wait()

    return fmt


@functools.lru_cache(maxsize=None)
def _build(batch: int, hist: int, vocab_pad: int):
    info = plsc.get_sparse_core_info()
    nc, ns = info.num_cores, info.num_subcores
    nw = nc * ns
    bpw = batch // nw
    assert batch % (nw * 128) == 0 and bpw % 128 == 0
    ntile = bpw // 128
    ncomp8 = EMBED_DIM // 8
    nblk = bpw // LANE
    assert hist % NBUF == 0
    n_groups = hist // NBUF

    mesh = plsc.VectorSubcoreMesh(core_axis_name="c", subcore_axis_name="s")

    @functools.partial(
        pl.kernel,
        mesh=mesh,
        out_type=jax.ShapeDtypeStruct(
            (hist, ncomp8, batch // 128, 8, 128), jnp.float32),
        scratch_types=[
            pltpu.VMEM((hist, bpw), jnp.int32),
            pltpu.VMEM((NBUF, bpw, EMBED_DIM), jnp.float32),
            pltpu.VMEM((NBUF, ncomp8, ntile + 1, 8, 129), jnp.float32),
            pltpu.SemaphoreType.DMA((NBUF,)),
            pltpu.SemaphoreType.DMA((NBUF,)),
        ],
        compiler_params=pltpu.CompilerParams(
            use_tc_tiling_on_sc=False, needs_layout_passes=False),
    )
    def grab(table_hbm, idxt_hbm, out_hbm, idx_v, rows_v, tbuf, gsem, osem):
        wid = lax.axis_index("s") * nc + lax.axis_index("c")
        b0 = wid * bpw
        bj0 = wid * ntile
        pltpu.sync_copy(idxt_hbm.at[:, pl.ds(b0, bpw)], idx_v)

        def gather(h, slot):
            return pltpu.make_async_copy(
                table_hbm.at[idx_v.at[h]], rows_v.at[slot], gsem.at[slot])

        def writeback(h, slot, c8):
            return pltpu.make_async_copy(
                tbuf.at[slot, c8, pl.ds(0, ntile), :, pl.ds(0, 128)],
                out_hbm.at[h, c8, pl.ds(bj0, ntile)],
                osem.at[slot])

        iota16 = jax.lax.iota(jnp.int32, LANE)
        cs_vec = iota16 % 8
        c8_vecs = [iota16 // 8 + 2 * half for half in range(2)]

        def transpose(slot):
            rows = rows_v.at[slot]
            tb = tbuf.at[slot]

            @plsc.parallel_loop(0, bpw, 1, unroll=8)
            def _(t):
                bjj_vec = jnp.full((LANE,), t // 128, jnp.int32)
                bl_vec = jnp.full((LANE,), t % 128, jnp.int32)
                for half in range(2):
                    vals = rows[t, pl.ds(half * LANE, LANE)]
                    plsc.store_scatter(
                        tb, [c8_vecs[half], bjj_vec, cs_vec, bl_vec], vals)

        for slot in range(NBUF):
            gather(slot, slot).start()

        def group(g, _):
            for slot in range(NBUF):
                h = g * NBUF + slot
                gather(h, slot).wait()

                @pl.when(h >= NBUF)
                def _():
                    for c8 in range(ncomp8):
                        writeback(h, slot, c8).wait()

                transpose(slot)
                gather(h + NBUF, slot).start()
                for c8 in range(ncomp8):
                    writeback(h, slot, c8).start()
            return 0

        lax.fori_loop(0, n_groups - 1, group, 0)

        for slot in range(NBUF):
            h = (n_groups - 1) * NBUF + slot
            gather(h, slot).wait()
            for c8 in range(ncomp8):
                writeback(h, slot, c8).wait()
            transpose(slot)
            for c8 in range(ncomp8):
                writeback(h, slot, c8).start()
        for slot in range(NBUF):
            for c8 in range(ncomp8):
                writeback(0, slot, c8).wait()

    return grab


def kernel(input, weight):
    b, h = input.shape
    vocab, d = weight.shape
    vocab_pad = -(-vocab // 128) * 128
    nj = vocab_pad // 128
    w4 = jnp.pad(weight, ((0, vocab_pad - vocab), (0, 0))).T.reshape(
        d // 8, 8, nj, 128).transpose(0, 2, 1, 3)
    wfmt = _build_fmt(vocab_pad)(w4)
    out5 = _build(b, h, vocab_pad)(wfmt, input.T)
    out = out5.transpose(2, 4, 0, 1, 3).reshape(b, h, EMBED_DIM)
    return out


# speedup vs baseline: 1.0234x; 1.0234x over previous
import functools

import jax
import jax.numpy as jnp
from jax import lax
from jax.experimental import pallas as pl
from jax.experimental.pallas import tpu as pltpu
from jax.experimental.pallas import tpu_sc as plsc

VOCAB = 1_000_000
EMBED_DIM = 32
NBUF = 2
LANE = 16


@functools.lru_cache(maxsize=None)
def _build_fmt(vocab_pad: int):
    info = plsc.get_sparse_core_info()
    nc, ns = info.num_cores, info.num_subcores
    nw = nc * ns
    nj = vocab_pad // 128
    ncomp8 = EMBED_DIM // 8
    jblk = 4
    nbuf = 3
    nfetch = -(-(-(-nj // jblk)) // nw)
    nfetch += (-nfetch) % nbuf
    n_groups = nfetch // nbuf
    toks = jblk * 128

    mesh = plsc.VectorSubcoreMesh(core_axis_name="c", subcore_axis_name="s")

    @functools.partial(
        pl.kernel,
        mesh=mesh,
        out_type=jax.ShapeDtypeStruct((vocab_pad, EMBED_DIM), jnp.float32),
        scratch_types=[
            pltpu.VMEM((nbuf, ncomp8, jblk, 8, 128), jnp.float32),
            pltpu.VMEM((nbuf, toks, EMBED_DIM + 1), jnp.float32),
            pltpu.SemaphoreType.DMA((nbuf,)),
            pltpu.SemaphoreType.DMA((nbuf,)),
        ],
        compiler_params=pltpu.CompilerParams(
            use_tc_tiling_on_sc=False, needs_layout_passes=False),
    )
    def fmt(w4_hbm, out_hbm, bin_v, obuf, isem, osem):
        wid = lax.axis_index("s") * nc + lax.axis_index("c")
        iota16 = jax.lax.iota(jnp.int32, LANE)

        def j_of(i):
            return jnp.minimum((wid * nfetch + i) * jblk, nj - jblk)

        def fetch(i, slot):
            return pltpu.make_async_copy(
                w4_hbm.at[:, pl.ds(j_of(i), jblk)], bin_v.at[slot],
                isem.at[slot])

        def writeback(i, slot):
            return pltpu.make_async_copy(
                obuf.at[slot, :, pl.ds(0, EMBED_DIM)],
                out_hbm.at[pl.ds(j_of(i) * 128, toks)],
                osem.at[slot])

        def transpose(slot):
            src = bin_v.at[slot]
            dst = obuf.at[slot]

            @plsc.parallel_loop(0, toks // LANE, 1, unroll=4)
            def _(q):
                jl = q // (128 // LANE)
                lb = q % (128 // LANE)
                ridx = q * LANE + iota16
                for c in range(EMBED_DIM):
                    cidx = jnp.full((LANE,), c, jnp.int32)
                    vals = src[c // 8, jl, c % 8, pl.ds(lb * LANE, LANE)]
                    plsc.store_scatter(dst, [ridx, cidx], vals)

        for slot in range(nbuf):
            fetch(slot, slot).start()

        def group(g, _):
            for slot in range(nbuf):
                i = g * nbuf + slot
                fetch(i, slot).wait()

                @pl.when(i >= nbuf)
                def _():
                    writeback(i, slot).wait()

                transpose(slot)
                fetch(i + nbuf, slot).start()
                writeback(i, slot).start()
            return 0

        lax.fori_loop(0, n_groups - 1, group, 0)

        for slot in range(nbuf):
            i = (n_groups - 1) * nbuf + slot
            fetch(i, slot).wait()
            writeback(i, slot).wait()
            transpose(slot)
            writeback(i, slot).start()
        for slot in range(nbuf):
            writeback(0, slot).wait()

    return fmt


@functools.lru_cache(maxsize=None)
def _build(batch: int, hist: int, vocab_pad: int):
    info = plsc.get_sparse_core_info()
    nc, ns = info.num_cores, info.num_subcores
    nw = nc * ns
    bpw = batch // nw
    assert batch % (nw * 128) == 0 and bpw % 128 == 0
    ntile = bpw // 128
    ncomp8 = EMBED_DIM // 8
    nblk = bpw // LANE
    assert hist % NBUF == 0
    n_groups = hist // NBUF

    mesh = plsc.VectorSubcoreMesh(core_axis_name="c", subcore_axis_name="s")

    @functools.partial(
        pl.kernel,
        mesh=mesh,
        out_type=jax.ShapeDtypeStruct(
            (hist, ncomp8, batch // 128, 8, 128), jnp.float32),
        scratch_types=[
            pltpu.VMEM((hist, bpw), jnp.int32),
            pltpu.VMEM((NBUF, bpw, EMBED_DIM), jnp.float32),
            pltpu.VMEM((NBUF, ncomp8, ntile + 1, 8, 129), jnp.float32),
            pltpu.SemaphoreType.DMA((NBUF,)),
            pltpu.SemaphoreType.DMA((NBUF,)),
        ],
        compiler_params=pltpu.CompilerParams(
            use_tc_tiling_on_sc=False, needs_layout_passes=False),
    )
    def grab(table_hbm, idxt_hbm, out_hbm, idx_v, rows_v, tbuf, gsem, osem):
        wid = lax.axis_index("s") * nc + lax.axis_index("c")
        b0 = wid * bpw
        bj0 = wid * ntile
        pltpu.sync_copy(idxt_hbm.at[:, pl.ds(b0, bpw)], idx_v)

        def gather(h, slot):
            return pltpu.make_async_copy(
                table_hbm.at[idx_v.at[h]], rows_v.at[slot], gsem.at[slot])

        def writeback(h, slot, c8):
            return pltpu.make_async_copy(
                tbuf.at[slot, c8, pl.ds(0, ntile), :, pl.ds(0, 128)],
                out_hbm.at[h, c8, pl.ds(bj0, ntile)],
                osem.at[slot])

        iota16 = jax.lax.iota(jnp.int32, LANE)
        cs_vec = iota16 % 8
        c8_vecs = [iota16 // 8 + 2 * half for half in range(2)]

        def transpose(slot):
            rows = rows_v.at[slot]
            tb = tbuf.at[slot]

            @plsc.parallel_loop(0, bpw, 1, unroll=8)
            def _(t):
                bjj_vec = jnp.full((LANE,), t // 128, jnp.int32)
                bl_vec = jnp.full((LANE,), t % 128, jnp.int32)
                for half in range(2):
                    vals = rows[t, pl.ds(half * LANE, LANE)]
                    plsc.store_scatter(
                        tb, [c8_vecs[half], bjj_vec, cs_vec, bl_vec], vals)

        for slot in range(NBUF):
            gather(slot, slot).start()

        def group(g, _):
            for slot in range(NBUF):
                h = g * NBUF + slot
                gather(h, slot).wait()

                @pl.when(h >= NBUF)
                def _():
                    for c8 in range(ncomp8):
                        writeback(h, slot, c8).wait()

                transpose(slot)
                gather(h + NBUF, slot).start()
                for c8 in range(ncomp8):
                    writeback(h, slot, c8).start()
            return 0

        lax.fori_loop(0, n_groups - 1, group, 0)

        for slot in range(NBUF):
            h = (n_groups - 1) * NBUF + slot
            gather(h, slot).wait()
            for c8 in range(ncomp8):
                writeback(h, slot, c8).wait()
            transpose(slot)
            for c8 in range(ncomp8):
                writeback(h, slot, c8).start()
        for slot in range(NBUF):
            for c8 in range(ncomp8):
                writeback(0, slot, c8).wait()

    return grab


def kernel(input, weight):
    b, h = input.shape
    vocab, d = weight.shape
    vocab_pad = -(-vocab // 128) * 128
    nj = vocab_pad // 128
    w4 = jnp.pad(weight, ((0, vocab_pad - vocab), (0, 0))).T.reshape(
        d // 8, 8, nj, 128).transpose(0, 2, 1, 3)
    wfmt = _build_fmt(vocab_pad)(w4)
    out5 = _build(b, h, vocab_pad)(wfmt, input.T)
    out = out5.transpose(2, 4, 0, 1, 3).reshape(b, h, EMBED_DIM)
    return out


# speedup vs baseline: 1.5133x; 1.4788x over previous
import functools

import jax
import jax.numpy as jnp
from jax import lax
from jax.experimental import pallas as pl
from jax.experimental.pallas import tpu as pltpu
from jax.experimental.pallas import tpu_sc as plsc

VOCAB = 1_000_000
EMBED_DIM = 32
NBUF = 2
LANE = 16


@functools.lru_cache(maxsize=None)
def _build_fmt(vocab_pad: int):
    info = plsc.get_sparse_core_info()
    nc, ns = info.num_cores, info.num_subcores
    nw = nc * ns
    nj = vocab_pad // 128
    ncomp8 = EMBED_DIM // 8
    jblk = 4
    nbuf = 3
    nfetch = -(-(-(-nj // jblk)) // nw)
    nfetch += (-nfetch) % nbuf
    n_groups = nfetch // nbuf
    toks = jblk * 128

    mesh = plsc.VectorSubcoreMesh(core_axis_name="c", subcore_axis_name="s")

    @functools.partial(
        pl.kernel,
        mesh=mesh,
        out_type=jax.ShapeDtypeStruct((vocab_pad, EMBED_DIM + 1), jnp.float32),
        scratch_types=[
            pltpu.VMEM((nbuf, ncomp8, jblk, 8, 128), jnp.float32),
            pltpu.VMEM((nbuf, toks, EMBED_DIM + 1), jnp.float32),
            pltpu.SemaphoreType.DMA((nbuf,)),
            pltpu.SemaphoreType.DMA((nbuf,)),
        ],
        compiler_params=pltpu.CompilerParams(
            use_tc_tiling_on_sc=False, needs_layout_passes=False),
    )
    def fmt(w4_hbm, out_hbm, bin_v, obuf, isem, osem):
        wid = lax.axis_index("s") * nc + lax.axis_index("c")
        iota16 = jax.lax.iota(jnp.int32, LANE)

        def j_of(i):
            return jnp.minimum((wid * nfetch + i) * jblk, nj - jblk)

        def fetch(i, slot):
            return pltpu.make_async_copy(
                w4_hbm.at[:, pl.ds(j_of(i), jblk)], bin_v.at[slot],
                isem.at[slot])

        def writeback(i, slot):
            return pltpu.make_async_copy(
                obuf.at[slot],
                out_hbm.at[pl.ds(j_of(i) * 128, toks)],
                osem.at[slot])

        def transpose(slot):
            src = bin_v.at[slot]
            dst = obuf.at[slot]

            @plsc.parallel_loop(0, toks // LANE, 1, unroll=4)
            def _(q):
                jl = q // (128 // LANE)
                lb = q % (128 // LANE)
                ridx = q * LANE + iota16
                for c in range(EMBED_DIM):
                    cidx = jnp.full((LANE,), c, jnp.int32)
                    vals = src[c // 8, jl, c % 8, pl.ds(lb * LANE, LANE)]
                    plsc.store_scatter(dst, [ridx, cidx], vals)

        for slot in range(nbuf):
            fetch(slot, slot).start()

        def group(g, _):
            for slot in range(nbuf):
                i = g * nbuf + slot
                fetch(i, slot).wait()

                @pl.when(i >= nbuf)
                def _():
                    writeback(i, slot).wait()

                transpose(slot)
                fetch(i + nbuf, slot).start()
                writeback(i, slot).start()
            return 0

        lax.fori_loop(0, n_groups - 1, group, 0)

        for slot in range(nbuf):
            i = (n_groups - 1) * nbuf + slot
            fetch(i, slot).wait()
            writeback(i, slot).wait()
            transpose(slot)
            writeback(i, slot).start()
        for slot in range(nbuf):
            writeback(0, slot).wait()

    return fmt


@functools.lru_cache(maxsize=None)
def _build(batch: int, hist: int, vocab_pad: int):
    info = plsc.get_sparse_core_info()
    nc, ns = info.num_cores, info.num_subcores
    nw = nc * ns
    bpw = batch // nw
    assert batch % (nw * 128) == 0 and bpw % 128 == 0
    ntile = bpw // 128
    ncomp8 = EMBED_DIM // 8
    nblk = bpw // LANE
    assert hist % NBUF == 0
    n_groups = hist // NBUF

    mesh = plsc.VectorSubcoreMesh(core_axis_name="c", subcore_axis_name="s")

    @functools.partial(
        pl.kernel,
        mesh=mesh,
        out_type=jax.ShapeDtypeStruct(
            (hist, ncomp8, batch // 128, 8, 128), jnp.float32),
        scratch_types=[
            pltpu.VMEM((hist, bpw), jnp.int32),
            pltpu.VMEM((NBUF, bpw, EMBED_DIM + 1), jnp.float32),
            pltpu.VMEM((NBUF, ncomp8, ntile + 1, 8, 129), jnp.float32),
            pltpu.SemaphoreType.DMA((NBUF,)),
            pltpu.SemaphoreType.DMA((NBUF,)),
        ],
        compiler_params=pltpu.CompilerParams(
            use_tc_tiling_on_sc=False, needs_layout_passes=False),
    )
    def grab(table_hbm, idxt_hbm, out_hbm, idx_v, rows_v, tbuf, gsem, osem):
        wid = lax.axis_index("s") * nc + lax.axis_index("c")
        b0 = wid * bpw
        bj0 = wid * ntile
        pltpu.sync_copy(idxt_hbm.at[:, pl.ds(b0, bpw)], idx_v)

        def gather(h, slot):
            return pltpu.make_async_copy(
                table_hbm.at[idx_v.at[h]], rows_v.at[slot], gsem.at[slot])

        def writeback(h, slot, c8):
            return pltpu.make_async_copy(
                tbuf.at[slot, c8, pl.ds(0, ntile), :, pl.ds(0, 128)],
                out_hbm.at[h, c8, pl.ds(bj0, ntile)],
                osem.at[slot])

        iota16 = jax.lax.iota(jnp.int32, LANE)
        cs_vec = iota16 % 8
        c8_vecs = [iota16 // 8 + 2 * half for half in range(2)]

        def transpose(slot):
            rows = rows_v.at[slot]
            tb = tbuf.at[slot]

            @plsc.parallel_loop(0, bpw, 1, unroll=8)
            def _(t):
                bjj_vec = jnp.full((LANE,), t // 128, jnp.int32)
                bl_vec = jnp.full((LANE,), t % 128, jnp.int32)
                for half in range(2):
                    vals = rows[t, pl.ds(half * LANE, LANE)]
                    plsc.store_scatter(
                        tb, [c8_vecs[half], bjj_vec, cs_vec, bl_vec], vals)

        for slot in range(NBUF):
            gather(slot, slot).start()

        def group(g, _):
            for slot in range(NBUF):
                h = g * NBUF + slot
                gather(h, slot).wait()

                @pl.when(h >= NBUF)
                def _():
                    for c8 in range(ncomp8):
                        writeback(h, slot, c8).wait()

                transpose(slot)
                gather(h + NBUF, slot).start()
                for c8 in range(ncomp8):
                    writeback(h, slot, c8).start()
            return 0

        lax.fori_loop(0, n_groups - 1, group, 0)

        for slot in range(NBUF):
            h = (n_groups - 1) * NBUF + slot
            gather(h, slot).wait()
            for c8 in range(ncomp8):
                writeback(h, slot, c8).wait()
            transpose(slot)
            for c8 in range(ncomp8):
                writeback(h, slot, c8).start()
        for slot in range(NBUF):
            for c8 in range(ncomp8):
                writeback(0, slot, c8).wait()

    return grab


def kernel(input, weight):
    b, h = input.shape
    vocab, d = weight.shape
    vocab_pad = -(-vocab // 128) * 128
    nj = vocab_pad // 128
    w4 = jnp.pad(weight, ((0, vocab_pad - vocab), (0, 0))).T.reshape(
        d // 8, 8, nj, 128).transpose(0, 2, 1, 3)
    wfmt = _build_fmt(vocab_pad)(w4)
    out5 = _build(b, h, vocab_pad)(wfmt, input.T)
    out = out5.transpose(2, 4, 0, 1, 3).reshape(b, h, EMBED_DIM)
    return out


# speedup vs baseline: 1.5636x; 1.0332x over previous
import functools

import jax
import jax.numpy as jnp
from jax import lax
from jax.experimental import pallas as pl
from jax.experimental.pallas import tpu as pltpu
from jax.experimental.pallas import tpu_sc as plsc

VOCAB = 1_000_000
EMBED_DIM = 32
NBUF = 2
LANE = 16


@functools.lru_cache(maxsize=None)
def _build_fmt(vocab_pad: int):
    info = plsc.get_sparse_core_info()
    nc, ns = info.num_cores, info.num_subcores
    nw = nc * ns
    nj = vocab_pad // 128
    ncomp8 = EMBED_DIM // 8
    jblk = 4
    nbuf = 3
    nfetch = -(-(-(-nj // jblk)) // nw)
    nfetch += (-nfetch) % nbuf
    n_groups = nfetch // nbuf
    toks = jblk * 128

    mesh = plsc.VectorSubcoreMesh(core_axis_name="c", subcore_axis_name="s")

    @functools.partial(
        pl.kernel,
        mesh=mesh,
        out_type=jax.ShapeDtypeStruct((vocab_pad, EMBED_DIM), jnp.float32),
        scratch_types=[
            pltpu.VMEM((nbuf, ncomp8, jblk + 1, 8, 129), jnp.float32),
            pltpu.VMEM((nbuf, toks, EMBED_DIM), jnp.float32),
            pltpu.SemaphoreType.DMA((nbuf,)),
            pltpu.SemaphoreType.DMA((nbuf,)),
        ],
        compiler_params=pltpu.CompilerParams(
            use_tc_tiling_on_sc=False, needs_layout_passes=False),
    )
    def fmt(w4_hbm, out_hbm, bin_v, obuf, isem, osem):
        wid = lax.axis_index("s") * nc + lax.axis_index("c")
        iota16 = jax.lax.iota(jnp.int32, LANE)

        def j_of(i):
            return jnp.minimum((wid * nfetch + i) * jblk, nj - jblk)

        def fetch(i, slot):
            return pltpu.make_async_copy(
                w4_hbm.at[:, pl.ds(j_of(i), jblk)],
                bin_v.at[slot, :, pl.ds(0, jblk), :, pl.ds(0, 128)],
                isem.at[slot])

        def writeback(i, slot):
            return pltpu.make_async_copy(
                obuf.at[slot],
                out_hbm.at[pl.ds(j_of(i) * 128, toks)],
                osem.at[slot])

        s_vec = iota16 % 8
        a_vecs = [iota16 // 8 + 2 * half for half in range(2)]

        def transpose(slot):
            src = bin_v.at[slot]

            @plsc.parallel_loop(0, toks, 1, unroll=8)
            def _(t):
                jl_vec = jnp.full((LANE,), t // 128, jnp.int32)
                l_vec = jnp.full((LANE,), t % 128, jnp.int32)
                for half in range(2):
                    vals = plsc.load_gather(
                        src, [a_vecs[half], jl_vec, s_vec, l_vec])
                    obuf[slot, t, pl.ds(half * LANE, LANE)] = vals

        for slot in range(nbuf):
            fetch(slot, slot).start()

        def group(g, _):
            for slot in range(nbuf):
                i = g * nbuf + slot
                fetch(i, slot).wait()

                @pl.when(i >= nbuf)
                def _():
                    writeback(i, slot).wait()

                transpose(slot)
                fetch(i + nbuf, slot).start()
                writeback(i, slot).start()
            return 0

        lax.fori_loop(0, n_groups - 1, group, 0)

        for slot in range(nbuf):
            i = (n_groups - 1) * nbuf + slot
            fetch(i, slot).wait()
            writeback(i, slot).wait()
            transpose(slot)
            writeback(i, slot).start()
        for slot in range(nbuf):
            writeback(0, slot).wait()

    return fmt


@functools.lru_cache(maxsize=None)
def _build(batch: int, hist: int, vocab_pad: int):
    info = plsc.get_sparse_core_info()
    nc, ns = info.num_cores, info.num_subcores
    nw = nc * ns
    bpw = batch // nw
    assert batch % (nw * 128) == 0 and bpw % 128 == 0
    ntile = bpw // 128
    ncomp8 = EMBED_DIM // 8
    nblk = bpw // LANE
    assert hist % NBUF == 0
    n_groups = hist // NBUF

    mesh = plsc.VectorSubcoreMesh(core_axis_name="c", subcore_axis_name="s")

    @functools.partial(
        pl.kernel,
        mesh=mesh,
        out_type=jax.ShapeDtypeStruct(
            (hist, ncomp8, batch // 128, 8, 128), jnp.float32),
        scratch_types=[
            pltpu.VMEM((hist, bpw), jnp.int32),
            pltpu.VMEM((NBUF, bpw, EMBED_DIM), jnp.float32),
            pltpu.VMEM((NBUF, ncomp8, ntile + 1, 8, 129), jnp.float32),
            pltpu.SemaphoreType.DMA((NBUF,)),
            pltpu.SemaphoreType.DMA((NBUF,)),
        ],
        compiler_params=pltpu.CompilerParams(
            use_tc_tiling_on_sc=False, needs_layout_passes=False),
    )
    def grab(table_hbm, idxt_hbm, out_hbm, idx_v, rows_v, tbuf, gsem, osem):
        wid = lax.axis_index("s") * nc + lax.axis_index("c")
        b0 = wid * bpw
        bj0 = wid * ntile
        pltpu.sync_copy(idxt_hbm.at[:, pl.ds(b0, bpw)], idx_v)

        def gather(h, slot):
            return pltpu.make_async_copy(
                table_hbm.at[idx_v.at[h]], rows_v.at[slot], gsem.at[slot])

        def writeback(h, slot, c8):
            return pltpu.make_async_copy(
                tbuf.at[slot, c8, pl.ds(0, ntile), :, pl.ds(0, 128)],
                out_hbm.at[h, c8, pl.ds(bj0, ntile)],
                osem.at[slot])

        iota16 = jax.lax.iota(jnp.int32, LANE)
        cs_vec = iota16 % 8
        c8_vecs = [iota16 // 8 + 2 * half for half in range(2)]

        def transpose(slot):
            rows = rows_v.at[slot]
            tb = tbuf.at[slot]

            @plsc.parallel_loop(0, bpw, 1, unroll=8)
            def _(t):
                bjj_vec = jnp.full((LANE,), t // 128, jnp.int32)
                bl_vec = jnp.full((LANE,), t % 128, jnp.int32)
                for half in range(2):
                    vals = rows[t, pl.ds(half * LANE, LANE)]
                    plsc.store_scatter(
                        tb, [c8_vecs[half], bjj_vec, cs_vec, bl_vec], vals)

        for slot in range(NBUF):
            gather(slot, slot).start()

        def group(g, _):
            for slot in range(NBUF):
                h = g * NBUF + slot
                gather(h, slot).wait()

                @pl.when(h >= NBUF)
                def _():
                    for c8 in range(ncomp8):
                        writeback(h, slot, c8).wait()

                transpose(slot)
                gather(h + NBUF, slot).start()
                for c8 in range(ncomp8):
                    writeback(h, slot, c8).start()
            return 0

        lax.fori_loop(0, n_groups - 1, group, 0)

        for slot in range(NBUF):
            h = (n_groups - 1) * NBUF + slot
            gather(h, slot).wait()
            for c8 in range(ncomp8):
                writeback(h, slot, c8).wait()
            transpose(slot)
            for c8 in range(ncomp8):
                writeback(h, slot, c8).start()
        for slot in range(NBUF):
            for c8 in range(ncomp8):
                writeback(0, slot, c8).wait()

    return grab


def kernel(input, weight):
    b, h = input.shape
    vocab, d = weight.shape
    vocab_pad = -(-vocab // 128) * 128
    nj = vocab_pad // 128
    w4 = jnp.pad(weight, ((0, vocab_pad - vocab), (0, 0))).T.reshape(
        d // 8, 8, nj, 128).transpose(0, 2, 1, 3)
    wfmt = _build_fmt(vocab_pad)(w4)
    out5 = _build(b, h, vocab_pad)(wfmt, input.T)
    out = out5.transpose(2, 4, 0, 1, 3).reshape(b, h, EMBED_DIM)
    return out


# speedup vs baseline: 1.5660x; 1.0016x over previous
import functools

import jax
import jax.numpy as jnp
from jax import lax
from jax.experimental import pallas as pl
from jax.experimental.pallas import tpu as pltpu
from jax.experimental.pallas import tpu_sc as plsc

VOCAB = 1_000_000
EMBED_DIM = 32
NBUF = 2
LANE = 16


@functools.lru_cache(maxsize=None)
def _build_fmt(vocab_pad: int):
    info = plsc.get_sparse_core_info()
    nc, ns = info.num_cores, info.num_subcores
    nw = nc * ns
    nj = vocab_pad // 128
    ncomp8 = EMBED_DIM // 8
    jblk = 4
    nbuf = 3
    nfetch = -(-(-(-nj // jblk)) // nw)
    nfetch += (-nfetch) % nbuf
    n_groups = nfetch // nbuf
    toks = jblk * 128

    mesh = plsc.VectorSubcoreMesh(core_axis_name="c", subcore_axis_name="s")

    @functools.partial(
        pl.kernel,
        mesh=mesh,
        out_type=jax.ShapeDtypeStruct((vocab_pad, EMBED_DIM), jnp.float32),
        scratch_types=[
            pltpu.VMEM((nbuf, ncomp8, jblk + 1, 8, 129), jnp.float32),
            pltpu.VMEM((nbuf, toks, EMBED_DIM), jnp.float32),
            pltpu.SemaphoreType.DMA((nbuf,)),
            pltpu.SemaphoreType.DMA((nbuf,)),
        ],
        compiler_params=pltpu.CompilerParams(
            use_tc_tiling_on_sc=False, needs_layout_passes=False),
    )
    def fmt(w4_hbm, out_hbm, bin_v, obuf, isem, osem):
        wid = lax.axis_index("s") * nc + lax.axis_index("c")
        iota16 = jax.lax.iota(jnp.int32, LANE)

        def j_of(i):
            return jnp.minimum((wid * nfetch + i) * jblk, nj - jblk)

        def fetch(i, slot):
            return pltpu.make_async_copy(
                w4_hbm.at[:, pl.ds(j_of(i), jblk)],
                bin_v.at[slot, :, pl.ds(0, jblk), :, pl.ds(0, 128)],
                isem.at[slot])

        def writeback(i, slot):
            return pltpu.make_async_copy(
                obuf.at[slot],
                out_hbm.at[pl.ds(j_of(i) * 128, toks)],
                osem.at[slot])

        s_vec = iota16 % 8
        a_vecs = [iota16 // 8 + 2 * half for half in range(2)]

        def transpose(slot):
            src = bin_v.at[slot]

            @plsc.parallel_loop(0, toks, 1, unroll=8)
            def _(t):
                jl_vec = jnp.full((LANE,), t // 128, jnp.int32)
                l_vec = jnp.full((LANE,), t % 128, jnp.int32)
                for half in range(2):
                    vals = plsc.load_gather(
                        src, [a_vecs[half], jl_vec, s_vec, l_vec])
                    obuf[slot, t, pl.ds(half * LANE, LANE)] = vals

        for slot in range(nbuf):
            fetch(slot, slot).start()

        def group(g, _):
            for slot in range(nbuf):
                i = g * nbuf + slot
                fetch(i, slot).wait()

                @pl.when(i >= nbuf)
                def _():
                    writeback(i, slot).wait()

                transpose(slot)
                fetch(i + nbuf, slot).start()
                writeback(i, slot).start()
            return 0

        lax.fori_loop(0, n_groups - 1, group, 0)

        for slot in range(nbuf):
            i = (n_groups - 1) * nbuf + slot
            fetch(i, slot).wait()
            writeback(i, slot).wait()
            transpose(slot)
            writeback(i, slot).start()
        for slot in range(nbuf):
            writeback(0, slot).wait()

    return fmt


@functools.lru_cache(maxsize=None)
def _build(batch: int, hist: int, vocab_pad: int):
    info = plsc.get_sparse_core_info()
    nc, ns = info.num_cores, info.num_subcores
    nw = nc * ns
    bpw = batch // nw
    assert batch % (nw * 128) == 0 and bpw % 128 == 0
    ntile = bpw // 128
    ncomp8 = EMBED_DIM // 8
    nblk = bpw // LANE
    assert hist % NBUF == 0
    n_groups = hist // NBUF

    mesh = plsc.VectorSubcoreMesh(core_axis_name="c", subcore_axis_name="s")

    @functools.partial(
        pl.kernel,
        mesh=mesh,
        out_type=jax.ShapeDtypeStruct(
            (hist, ncomp8, batch // 128, 8, 128), jnp.float32),
        scratch_types=[
            pltpu.VMEM((hist, bpw), jnp.int32),
            pltpu.VMEM((NBUF, bpw, EMBED_DIM), jnp.float32),
            pltpu.VMEM((NBUF, ncomp8, ntile + 1, 8, 129), jnp.float32),
            pltpu.SemaphoreType.DMA((NBUF,)),
            pltpu.SemaphoreType.DMA((NBUF,)),
        ],
        compiler_params=pltpu.CompilerParams(
            use_tc_tiling_on_sc=False, needs_layout_passes=False),
    )
    def grab(table_hbm, idxt_hbm, out_hbm, idx_v, rows_v, tbuf, gsem, osem):
        wid = lax.axis_index("s") * nc + lax.axis_index("c")
        b0 = wid * bpw
        bj0 = wid * ntile
        pltpu.sync_copy(idxt_hbm.at[:, pl.ds(b0, bpw)], idx_v)

        def gather(h, slot):
            return pltpu.make_async_copy(
                table_hbm.at[idx_v.at[h]], rows_v.at[slot], gsem.at[slot])

        def writeback(h, slot):
            return pltpu.make_async_copy(
                tbuf.at[slot, :, pl.ds(0, ntile), :, pl.ds(0, 128)],
                out_hbm.at[h, :, pl.ds(bj0, ntile)],
                osem.at[slot])

        iota16 = jax.lax.iota(jnp.int32, LANE)
        cs_vec = iota16 % 8
        c8_vecs = [iota16 // 8 + 2 * half for half in range(2)]

        def transpose(slot):
            rows = rows_v.at[slot]
            tb = tbuf.at[slot]

            @plsc.parallel_loop(0, bpw, 1, unroll=8)
            def _(t):
                bjj_vec = jnp.full((LANE,), t // 128, jnp.int32)
                bl_vec = jnp.full((LANE,), t % 128, jnp.int32)
                for half in range(2):
                    vals = rows[t, pl.ds(half * LANE, LANE)]
                    plsc.store_scatter(
                        tb, [c8_vecs[half], bjj_vec, cs_vec, bl_vec], vals)

        for slot in range(NBUF):
            gather(slot, slot).start()

        def group(g, _):
            for slot in range(NBUF):
                h = g * NBUF + slot
                gather(h, slot).wait()

                @pl.when(h >= NBUF)
                def _():
                    writeback(h, slot).wait()

                transpose(slot)
                gather(h + NBUF, slot).start()
                writeback(h, slot).start()
            return 0

        lax.fori_loop(0, n_groups - 1, group, 0)

        for slot in range(NBUF):
            h = (n_groups - 1) * NBUF + slot
            gather(h, slot).wait()
            writeback(h, slot).wait()
            transpose(slot)
            writeback(h, slot).start()
        for slot in range(NBUF):
            writeback(0, slot).wait()

    return grab


def kernel(input, weight):
    b, h = input.shape
    vocab, d = weight.shape
    vocab_pad = -(-vocab // 128) * 128
    nj = vocab_pad // 128
    w4 = jnp.pad(weight, ((0, vocab_pad - vocab), (0, 0))).T.reshape(
        d // 8, 8, nj, 128).transpose(0, 2, 1, 3)
    wfmt = _build_fmt(vocab_pad)(w4)
    out5 = _build(b, h, vocab_pad)(wfmt, input.T)
    out = out5.transpose(2, 4, 0, 1, 3).reshape(b, h, EMBED_DIM)
    return out
